# baseline (speedup 1.0000x reference)
import functools

import jax
import jax.numpy as jnp
from jax import lax
from jax.experimental import pallas as pl
from jax.experimental.pallas import tpu as pltpu
from jax.experimental.pallas import tpu_sc as plsc

SIGMA = 0.05
K_TOP = 49
GROUP = 8
D_PAD = 208
_INT_MIN = -2147483648
_POS_MASK = 0x7FFFFFFF


def _tc_body(x_ref, noise_ref, out_ref, *, ns, d, k, g):
    x = x_ref[:]
    noise = noise_ref[:]
    v = (x + SIGMA * noise).reshape(g * ns, d)

    vt = jnp.transpose(v)
    rows = g * ns

    bits = lax.bitcast_convert_type(vt, jnp.int32)
    key = jnp.where(bits >= 0, bits, bits ^ jnp.int32(_POS_MASK))

    def sel_step(i, t):
        p = 31 - i
        t_cand = t + (jnp.int32(1) << p)
        cnt = jnp.sum((key >= t_cand).astype(jnp.int32), axis=0, keepdims=True)
        return jnp.where(cnt >= k, t_cand, t)

    t0 = jnp.full((1, rows), _INT_MIN, jnp.int32)
    t = lax.fori_loop(0, 32, sel_step, t0)

    m_gt = key > t
    cnt_gt = jnp.sum(m_gt.astype(jnp.int32), axis=0, keepdims=True)
    need = (k - cnt_gt).astype(jnp.float32)
    eq = key == t

    row_i = lax.broadcasted_iota(jnp.int32, (d, d), 0)
    col_i = lax.broadcasted_iota(jnp.int32, (d, d), 1)
    low_strict = (row_i > col_i).astype(jnp.float32)
    low_incl = (row_i >= col_i).astype(jnp.float32)

    eq_rank = jnp.dot(low_strict, eq.astype(jnp.float32),
                      preferred_element_type=jnp.float32)
    m = m_gt | (eq & (eq_rank < need))

    s = jnp.dot(low_incl, m.astype(jnp.float32),
                preferred_element_type=jnp.float32)
    d_iota = lax.broadcasted_iota(jnp.int32, (d, rows), 0).astype(jnp.float32)
    c = jnp.where(m, s - 1.0, k + d_iota - s)
    ci_t = c.astype(jnp.int32)

    ci = jnp.transpose(ci_t)
    lane = lax.broadcasted_iota(jnp.int32, (rows, d), 1)
    flat = ci * D_PAD + lane
    pad = lax.broadcasted_iota(jnp.int32, (rows, D_PAD - d), 1) + d
    out_ref[:] = jnp.concatenate([flat, pad], axis=1).reshape(g, ns, D_PAD)


def _tc_indices(x, noise):
    b, d = x.shape
    ns = noise.shape[1]
    g = GROUP
    x3 = x.reshape(b, 1, d)
    body = functools.partial(_tc_body, ns=ns, d=d, k=K_TOP, g=g)
    return pl.pallas_call(
        body,
        grid=(b // g,),
        in_specs=[
            pl.BlockSpec((g, 1, d), lambda i: (i, 0, 0)),
            pl.BlockSpec((g, ns, d), lambda i: (i, 0, 0)),
        ],
        out_specs=pl.BlockSpec((g, ns, D_PAD), lambda i: (i, 0, 0)),
        out_shape=jax.ShapeDtypeStruct((b, ns, D_PAD), jnp.int32),
    )(x3, noise)


def _sc_histogram(ci, b, ns, d):
    inv_ns = 1.0 / ns
    nchunk = D_PAD // 16
    runroll = 8

    @functools.partial(
        pl.kernel,
        mesh=plsc.VectorSubcoreMesh(core_axis_name="c", subcore_axis_name="s"),
        out_type=jax.ShapeDtypeStruct((b, d * D_PAD), jnp.float32),
        scratch_types=[
            pltpu.VMEM((ns * D_PAD,), jnp.int32),
            pltpu.VMEM((d * D_PAD,), jnp.float32),
            pltpu.SemaphoreType.DMA,
        ],
        compiler_params=pltpu.CompilerParams(needs_layout_passes=False),
    )
    def run(ci_hbm, out_hbm, rows_v, acc, sem):
        cid = lax.axis_index("c")
        sid = lax.axis_index("s")
        my_b = cid * (b // 2) + sid

        @pl.when(sid < b // 2)
        def _():
            cp = pltpu.async_copy(ci_hbm.at[my_b], rows_v, sem)

            zero = jnp.zeros((16,), jnp.float32)

            def _zero(r, carry):
                for ch in range(nchunk):
                    off = pl.multiple_of(r * D_PAD + ch * 16, 16)
                    acc[pl.ds(off, 16)] = zero
                return carry

            lax.fori_loop(0, d, _zero, 0)

            cp.wait()
            inv = jnp.full((16,), inv_ns, jnp.float32)

            def _scatter(r, carry):
                base = r * (runroll * D_PAD)
                for u in range(runroll * nchunk):
                    off = pl.multiple_of(base + u * 16, 16)
                    plsc.addupdate_scatter(acc, [rows_v[pl.ds(off, 16)]], inv)
                return carry

            lax.fori_loop(0, ns // runroll, _scatter, 0)

            pltpu.sync_copy(acc, out_hbm.at[my_b])

    return run(ci.reshape(b, ns * D_PAD))


def kernel(x, noise):
    b, d = x.shape
    ns = noise.shape[1]
    h = b // 2
    ci0 = _tc_indices(x[:h], noise[:h])
    out0 = _sc_histogram(ci0, h, ns, d)
    ci1 = _tc_indices(x[h:], noise[h:])
    out1 = _sc_histogram(ci1, h, ns, d)
    out = jnp.concatenate([out0, out1], axis=0).reshape(b, d, D_PAD)
    return out[:, :K_TOP, :d], out[:, K_TOP:, :d]


# speedup vs baseline: 1.2042x; 1.2042x over previous
import functools

import jax
import jax.numpy as jnp
from jax import lax
from jax.experimental import pallas as pl
from jax.experimental.pallas import tpu as pltpu
from jax.experimental.pallas import tpu_sc as plsc

SIGMA = 0.05
K_TOP = 49
GROUP = 8
D_PAD = 208
_INT_MIN = -2147483648
_POS_MASK = 0x7FFFFFFF


def _tc_body(x_ref, noise_ref, out_ref, *, ns, d, k, g):
    x = x_ref[:]
    noise = noise_ref[:]
    v = (x + SIGMA * noise).reshape(g * ns, d)

    vt = jnp.transpose(v)
    rows = g * ns

    bits = lax.bitcast_convert_type(vt, jnp.int32)
    key = jnp.where(bits >= 0, bits, bits ^ jnp.int32(_POS_MASK))

    def sel_step(i, t):
        p = 31 - i
        t_cand = t + (jnp.int32(1) << p)
        cnt = jnp.sum((key >= t_cand).astype(jnp.int32), axis=0, keepdims=True)
        return jnp.where(cnt >= k, t_cand, t)

    t0 = jnp.full((1, rows), _INT_MIN, jnp.int32)
    t = lax.fori_loop(0, 32, sel_step, t0)

    m_gt = key > t
    cnt_gt = jnp.sum(m_gt.astype(jnp.int32), axis=0, keepdims=True)
    need = (k - cnt_gt).astype(jnp.float32)
    eq = key == t

    row_i = lax.broadcasted_iota(jnp.int32, (d, d), 0)
    col_i = lax.broadcasted_iota(jnp.int32, (d, d), 1)
    low_strict = (row_i > col_i).astype(jnp.float32)
    low_incl = (row_i >= col_i).astype(jnp.float32)

    eq_rank = jnp.dot(low_strict, eq.astype(jnp.float32),
                      preferred_element_type=jnp.float32)
    m = m_gt | (eq & (eq_rank < need))

    s = jnp.dot(low_incl, m.astype(jnp.float32),
                preferred_element_type=jnp.float32)
    d_iota = lax.broadcasted_iota(jnp.int32, (d, rows), 0).astype(jnp.float32)
    c = jnp.where(m, s - 1.0, k + d_iota - s)
    ci_t = c.astype(jnp.int32)

    ci = jnp.transpose(ci_t)
    lane = lax.broadcasted_iota(jnp.int32, (rows, d), 1)
    flat = ci * d + lane
    pad = lax.broadcasted_iota(jnp.int32, (rows, D_PAD - d), 1) + d * d
    out_ref[:] = jnp.concatenate([flat, pad], axis=1).reshape(g, ns, D_PAD)


def _tc_indices(x, noise):
    b, d = x.shape
    ns = noise.shape[1]
    g = GROUP
    x3 = x.reshape(b, 1, d)
    body = functools.partial(_tc_body, ns=ns, d=d, k=K_TOP, g=g)
    return pl.pallas_call(
        body,
        grid=(b // g,),
        in_specs=[
            pl.BlockSpec((g, 1, d), lambda i: (i, 0, 0)),
            pl.BlockSpec((g, ns, d), lambda i: (i, 0, 0)),
        ],
        out_specs=pl.BlockSpec((g, ns, D_PAD), lambda i: (i, 0, 0)),
        out_shape=jax.ShapeDtypeStruct((b, ns, D_PAD), jnp.int32),
    )(x3, noise)


def _sc_histogram(ci, b, ns, d, k):
    inv_ns = 1.0 / ns
    nchunk = D_PAD // 16
    runroll = 8
    acc_len = -(-(d * d + (D_PAD - d)) // 256) * 256
    zchunks = acc_len // 256

    @functools.partial(
        pl.kernel,
        mesh=plsc.VectorSubcoreMesh(core_axis_name="c", subcore_axis_name="s"),
        out_type=jax.ShapeDtypeStruct(
            (b, -(-(d * d + (D_PAD - d)) // 256) * 256), jnp.float32),
        scratch_types=[
            pltpu.VMEM((ns * D_PAD,), jnp.int32),
            pltpu.VMEM((acc_len,), jnp.float32),
            pltpu.SemaphoreType.DMA,
        ],
        compiler_params=pltpu.CompilerParams(needs_layout_passes=False),
    )
    def run(ci_hbm, out_hbm, rows_v, acc, sem):
        cid = lax.axis_index("c")
        sid = lax.axis_index("s")
        my_b = cid * (b // 2) + sid

        @pl.when(sid < b // 2)
        def _():
            cp = pltpu.async_copy(ci_hbm.at[my_b], rows_v, sem)

            zero = jnp.zeros((16,), jnp.float32)

            def _zero(r, carry):
                for ch in range(16):
                    off = pl.multiple_of(r * 256 + ch * 16, 16)
                    acc[pl.ds(off, 16)] = zero
                return carry

            lax.fori_loop(0, zchunks, _zero, 0)

            cp.wait()
            inv = jnp.full((16,), inv_ns, jnp.float32)

            def _scatter(r, carry):
                base = r * (runroll * D_PAD)
                for u in range(runroll * nchunk):
                    off = pl.multiple_of(base + u * 16, 16)
                    plsc.addupdate_scatter(acc, [rows_v[pl.ds(off, 16)]], inv)
                return carry

            lax.fori_loop(0, ns // runroll, _scatter, 0)

            pltpu.sync_copy(acc, out_hbm.at[my_b])

    return run(ci.reshape(b, ns * D_PAD))


def kernel(x, noise):
    b, d = x.shape
    ns = noise.shape[1]
    k = K_TOP
    ci = _tc_indices(x, noise)
    out = _sc_histogram(ci, b, ns, d, k)
    return (out[:, :k * d].reshape(b, k, d),
            out[:, k * d:d * d].reshape(b, d - k, d))


# speedup vs baseline: 1.2368x; 1.0271x over previous
import functools

import jax
import jax.numpy as jnp
from jax import lax
from jax.experimental import pallas as pl
from jax.experimental.pallas import tpu as pltpu
from jax.experimental.pallas import tpu_sc as plsc

SIGMA = 0.05
K_TOP = 49
GROUP = 8
D_PAD = 208
_INT_MIN = -2147483648
_POS_MASK = 0x7FFFFFFF


def _tc_body(x_ref, noise_ref, out_ref, *, ns, d, k, g):
    x = x_ref[:]
    noise = noise_ref[:]
    v = (x + SIGMA * noise).reshape(g * ns, d)

    vt = jnp.transpose(v)
    rows = g * ns

    bits = lax.bitcast_convert_type(vt, jnp.int32)
    key = jnp.where(bits >= 0, bits, bits ^ jnp.int32(_POS_MASK))

    def sel_step(i, t):
        p = 31 - i
        t_cand = t + (jnp.int32(1) << p)
        cnt = jnp.sum((key >= t_cand).astype(jnp.int32), axis=0, keepdims=True)
        return jnp.where(cnt >= k, t_cand, t)

    t0 = jnp.full((1, rows), _INT_MIN, jnp.int32)
    t = lax.fori_loop(0, 32, sel_step, t0)

    m_gt = key > t
    cnt_gt = jnp.sum(m_gt.astype(jnp.int32), axis=0, keepdims=True)
    need = (k - cnt_gt).astype(jnp.float32)
    eq = key == t

    row_i = lax.broadcasted_iota(jnp.int32, (d, d), 0)
    col_i = lax.broadcasted_iota(jnp.int32, (d, d), 1)
    low_strict = (row_i > col_i).astype(jnp.float32)
    low_incl = (row_i >= col_i).astype(jnp.float32)

    eq_rank = jnp.dot(low_strict, eq.astype(jnp.float32),
                      preferred_element_type=jnp.float32)
    m = m_gt | (eq & (eq_rank < need))

    s = jnp.dot(low_incl, m.astype(jnp.float32),
                preferred_element_type=jnp.float32)
    d_iota = lax.broadcasted_iota(jnp.int32, (d, rows), 0).astype(jnp.float32)
    c = jnp.where(m, s - 1.0, k + d_iota - s)
    ci_t = c.astype(jnp.int32)

    ci = jnp.transpose(ci_t)
    lane = lax.broadcasted_iota(jnp.int32, (rows, d), 1)
    flat = ci * D_PAD + lane
    pad = lax.broadcasted_iota(jnp.int32, (rows, D_PAD - d), 1) + d
    out_ref[:] = jnp.concatenate([flat, pad], axis=1).reshape(g, ns, D_PAD)


def _tc_indices(x, noise):
    b, d = x.shape
    ns = noise.shape[1]
    g = GROUP
    x3 = x.reshape(b, 1, d)
    body = functools.partial(_tc_body, ns=ns, d=d, k=K_TOP, g=g)
    return pl.pallas_call(
        body,
        grid=(b // g,),
        in_specs=[
            pl.BlockSpec((g, 1, d), lambda i: (i, 0, 0)),
            pl.BlockSpec((g, ns, d), lambda i: (i, 0, 0)),
        ],
        out_specs=pl.BlockSpec((g, ns, D_PAD), lambda i: (i, 0, 0)),
        out_shape=jax.ShapeDtypeStruct((b, ns, D_PAD), jnp.int32),
    )(x3, noise)


def _sc_histogram(ci, b, ns, d, k):
    inv_ns = 1.0 / ns
    nchunk = D_PAD // 16
    runroll = 8

    @functools.partial(
        pl.kernel,
        mesh=plsc.VectorSubcoreMesh(core_axis_name="c", subcore_axis_name="s"),
        out_type=jax.ShapeDtypeStruct((b, d * D_PAD), jnp.float32),
        scratch_types=[
            pltpu.VMEM((ns * D_PAD,), jnp.int32),
            pltpu.VMEM((d * D_PAD,), jnp.float32),
            pltpu.SemaphoreType.DMA,
        ],
        compiler_params=pltpu.CompilerParams(needs_layout_passes=False),
    )
    def run(ci_hbm, out_hbm, rows_v, acc, sem):
        cid = lax.axis_index("c")
        sid = lax.axis_index("s")
        my_b = cid * (b // 2) + sid

        @pl.when(sid < b // 2)
        def _():
            cp = pltpu.async_copy(ci_hbm.at[my_b], rows_v, sem)

            zero = jnp.zeros((16,), jnp.float32)

            def _zero(r, carry):
                for ch in range(nchunk):
                    off = pl.multiple_of(r * D_PAD + ch * 16, 16)
                    acc[pl.ds(off, 16)] = zero
                return carry

            lax.fori_loop(0, d, _zero, 0)

            cp.wait()
            inv = jnp.full((16,), inv_ns, jnp.float32)

            def _scatter(r, carry):
                base = r * (runroll * D_PAD)
                for u in range(runroll * nchunk):
                    off = pl.multiple_of(base + u * 16, 16)
                    plsc.addupdate_scatter(acc, [rows_v[pl.ds(off, 16)]], inv)
                return carry

            lax.fori_loop(0, ns // runroll, _scatter, 0)

            pltpu.sync_copy(acc, out_hbm.at[my_b])

    return run(ci.reshape(b, ns * D_PAD))


def kernel(x, noise):
    b, d = x.shape
    ns = noise.shape[1]
    k = K_TOP
    ci = _tc_indices(x, noise)
    out = _sc_histogram(ci, b, ns, d, k).reshape(b, d, D_PAD)
    return out[:, :k, :d], out[:, k:, :d]
